# baseline (speedup 1.0000x reference)
import functools

import jax
import jax.numpy as jnp
from jax import lax
from jax.experimental import pallas as pl
from jax.experimental.pallas import tpu as pltpu

T, D, E, K, F, FS = 2048, 2048, 8, 2, 1024, 4096
BLK = 128
NB = (T * K) // BLK + E
NROWS = NB * BLK
TB = 128


def _router_body(x_ref, rw_ref, w_ref, pos_ref, be_ref):
    xf = x_ref[...]
    rw = rw_ref[...]
    logits = lax.dot_general(xf, rw, (((1,), (1,)), ((), ())),
                             preferred_element_type=jnp.float32)
    m = jnp.max(logits, axis=1, keepdims=True)
    p = jnp.exp(logits - m)
    probs = p / jnp.sum(p, axis=1, keepdims=True)

    eids = lax.broadcasted_iota(jnp.int32, (T, E), 1)
    v1 = jnp.max(probs, axis=1, keepdims=True)
    i1 = jnp.min(jnp.where(probs == v1, eids, E), axis=1, keepdims=True)
    probs2 = jnp.where(eids == i1, -1.0, probs)
    v2 = jnp.max(probs2, axis=1, keepdims=True)
    i2 = jnp.min(jnp.where(probs2 == v2, eids, E), axis=1, keepdims=True)
    s = v1 + v2
    w1 = v1 / s
    w2 = v2 / s

    ind = jnp.where(eids == i1, 1.0, 0.0) + jnp.where(eids == i2, 1.0, 0.0)

    c = ind
    shift = 1
    while shift < T:
        c = c + jnp.concatenate(
            [jnp.zeros((shift, E), jnp.float32), c[: T - shift, :]], axis=0)
        shift *= 2
    c_excl = c - ind
    totals = c[T - 1:T, :]
    nb_e = jnp.floor((totals + (BLK - 1)) / BLK)

    starts = []
    ends = []
    acc = jnp.zeros((1, 1), jnp.float32)
    for e in range(E):
        starts.append(acc)
        acc = acc + nb_e[:, e:e + 1]
        ends.append(acc)

    pos1 = jnp.zeros((T, 1), jnp.float32)
    pos2 = jnp.zeros((T, 1), jnp.float32)
    for e in range(E):
        base = starts[e] * BLK
        pos1 = pos1 + jnp.where(i1 == e, base + c_excl[:, e:e + 1], 0.0)
        pos2 = pos2 + jnp.where(i2 == e, base + c_excl[:, e:e + 1], 0.0)

    b_iota = lax.broadcasted_iota(jnp.int32, (1, NB), 1)
    be = jnp.zeros((1, NB), jnp.int32)
    for e in range(E):
        be = be + jnp.where(b_iota >= ends[e].astype(jnp.int32), 1, 0)
    be_ref[...] = jnp.minimum(be, E - 1)

    w_ref[...] = jnp.concatenate([w1, w2], axis=1)
    pos_ref[...] = jnp.concatenate([pos1, pos2], axis=1).astype(jnp.int32)


def _router(x, router_w):
    return pl.pallas_call(
        _router_body,
        out_shape=(
            jax.ShapeDtypeStruct((T, K), jnp.float32),
            jax.ShapeDtypeStruct((T, K), jnp.int32),
            jax.ShapeDtypeStruct((1, NB), jnp.int32),
        ),
    )(x, router_w)


def _moe_body(be_ref, xs_ref, ws_ref, wg_ref, wu_ref, wd_ref, y_ref):
    xb = xs_ref[...].astype(jnp.bfloat16)
    wg = wg_ref[0].astype(jnp.bfloat16)
    wu = wu_ref[0].astype(jnp.bfloat16)
    wd = wd_ref[0].astype(jnp.bfloat16)
    g = lax.dot_general(xb, wg, (((1,), (1,)), ((), ())),
                        preferred_element_type=jnp.float32)
    u = lax.dot_general(xb, wu, (((1,), (1,)), ((), ())),
                        preferred_element_type=jnp.float32)
    h = (g * jax.nn.sigmoid(g) * u).astype(jnp.bfloat16)
    y = lax.dot_general(h, wd, (((1,), (1,)), ((), ())),
                        preferred_element_type=jnp.float32)
    y_ref[...] = y * ws_ref[:, 0:1]


def _moe(block_expert, xs, ws, w_gate, w_up, w_down):
    grid_spec = pltpu.PrefetchScalarGridSpec(
        num_scalar_prefetch=1,
        grid=(NB,),
        in_specs=[
            pl.BlockSpec((BLK, D), lambda i, be: (i, 0)),
            pl.BlockSpec((BLK, 16), lambda i, be: (i, 0)),
            pl.BlockSpec((1, F, D), lambda i, be: (be[0, i], 0, 0)),
            pl.BlockSpec((1, F, D), lambda i, be: (be[0, i], 0, 0)),
            pl.BlockSpec((1, D, F), lambda i, be: (be[0, i], 0, 0)),
        ],
        out_specs=pl.BlockSpec((BLK, D), lambda i, be: (i, 0)),
    )
    return pl.pallas_call(
        _moe_body,
        grid_spec=grid_spec,
        out_shape=jax.ShapeDtypeStruct((NROWS, D), jnp.float32),
    )(block_expert, xs, ws, w_gate, w_up, w_down)


def _shared_body(x_ref, sg_ref, su_ref, sd_ref, o_ref):
    xb = x_ref[...]
    g = lax.dot_general(xb, sg_ref[...], (((1,), (1,)), ((), ())),
                        preferred_element_type=jnp.float32)
    u = lax.dot_general(xb, su_ref[...], (((1,), (1,)), ((), ())),
                        preferred_element_type=jnp.float32)
    h = (g * jax.nn.sigmoid(g) * u).astype(jnp.bfloat16)
    o_ref[...] = lax.dot_general(h, sd_ref[...], (((1,), (1,)), ((), ())),
                                 preferred_element_type=jnp.float32)


def _shared(x_bf, sg_bf, su_bf, sd_bf):
    ntb = T // TB
    return pl.pallas_call(
        _shared_body,
        grid=(ntb,),
        in_specs=[
            pl.BlockSpec((TB, D), lambda i: (i, 0)),
            pl.BlockSpec((FS, D), lambda i: (0, 0)),
            pl.BlockSpec((FS, D), lambda i: (0, 0)),
            pl.BlockSpec((D, FS), lambda i: (0, 0)),
        ],
        out_specs=pl.BlockSpec((TB, D), lambda i: (i, 0)),
        out_shape=jax.ShapeDtypeStruct((T, D), jnp.float32),
    )(x_bf, sg_bf, su_bf, sd_bf)


def kernel(x, router_w, w_gate, w_up, w_down, shared_gate, shared_up,
           shared_down):
    w, pos, block_expert = _router(x, router_w)

    tok = jnp.arange(T * K, dtype=jnp.int32) // K
    pos_flat = pos.reshape(-1)
    xs = jnp.zeros((NROWS, D), jnp.float32).at[pos_flat].set(x[tok])
    w_rows = jnp.broadcast_to(w.reshape(-1)[:, None], (T * K, 16))
    ws = jnp.zeros((NROWS, 16), jnp.float32).at[pos_flat].set(w_rows)

    ys = _moe(block_expert, xs, ws, w_gate, w_up, w_down)
    sh = _shared(x.astype(jnp.bfloat16), shared_gate.astype(jnp.bfloat16),
                 shared_up.astype(jnp.bfloat16), shared_down.astype(jnp.bfloat16))

    return sh + ys[pos[:, 0]] + ys[pos[:, 1]]


# speedup vs baseline: 1.0576x; 1.0576x over previous
import functools

import jax
import jax.numpy as jnp
from jax import lax
from jax.experimental import pallas as pl
from jax.experimental.pallas import tpu as pltpu
from jax.experimental.pallas import tpu_sc as plsc

T, D, E, K, F, FS = 2048, 2048, 8, 2, 1024, 4096
BLK = 128
NB = (T * K) // BLK + E
NROWS = NB * BLK
TB = 128


def _router_body(x_ref, rw_ref, w_ref, pos_ref, be_ref):
    xf = x_ref[...]
    rw = rw_ref[...]
    logits = lax.dot_general(xf, rw, (((1,), (1,)), ((), ())),
                             preferred_element_type=jnp.float32)
    m = jnp.max(logits, axis=1, keepdims=True)
    p = jnp.exp(logits - m)
    probs = p / jnp.sum(p, axis=1, keepdims=True)

    eids = lax.broadcasted_iota(jnp.int32, (T, E), 1)
    v1 = jnp.max(probs, axis=1, keepdims=True)
    i1 = jnp.min(jnp.where(probs == v1, eids, E), axis=1, keepdims=True)
    probs2 = jnp.where(eids == i1, -1.0, probs)
    v2 = jnp.max(probs2, axis=1, keepdims=True)
    i2 = jnp.min(jnp.where(probs2 == v2, eids, E), axis=1, keepdims=True)
    s = v1 + v2
    w1 = v1 / s
    w2 = v2 / s

    ind = jnp.where(eids == i1, 1.0, 0.0) + jnp.where(eids == i2, 1.0, 0.0)

    c = ind
    shift = 1
    while shift < T:
        c = c + jnp.concatenate(
            [jnp.zeros((shift, E), jnp.float32), c[: T - shift, :]], axis=0)
        shift *= 2
    c_excl = c - ind
    totals = c[T - 1:T, :]
    nb_e = jnp.floor((totals + (BLK - 1)) / BLK)

    starts = []
    ends = []
    acc = jnp.zeros((1, 1), jnp.float32)
    for e in range(E):
        starts.append(acc)
        acc = acc + nb_e[:, e:e + 1]
        ends.append(acc)

    pos1 = jnp.zeros((T, 1), jnp.float32)
    pos2 = jnp.zeros((T, 1), jnp.float32)
    for e in range(E):
        base = starts[e] * BLK
        pos1 = pos1 + jnp.where(i1 == e, base + c_excl[:, e:e + 1], 0.0)
        pos2 = pos2 + jnp.where(i2 == e, base + c_excl[:, e:e + 1], 0.0)

    b_iota = lax.broadcasted_iota(jnp.int32, (1, NB), 1)
    be = jnp.zeros((1, NB), jnp.int32)
    for e in range(E):
        be = be + jnp.where(b_iota >= ends[e].astype(jnp.int32), 1, 0)
    be_ref[...] = jnp.minimum(be, E - 1)

    w_ref[...] = jnp.concatenate([w1, w2], axis=1)
    pos_ref[...] = jnp.concatenate([pos1, pos2], axis=1).astype(jnp.int32)


def _router(x, router_w):
    return pl.pallas_call(
        _router_body,
        out_shape=(
            jax.ShapeDtypeStruct((T, K), jnp.float32),
            jax.ShapeDtypeStruct((T, K), jnp.int32),
            jax.ShapeDtypeStruct((1, NB), jnp.int32),
        ),
    )(x, router_w)


def _moe_body(be_ref, xs_ref, ws_ref, wg_ref, wu_ref, wd_ref, y_ref):
    xb = xs_ref[...].astype(jnp.bfloat16)
    wg = wg_ref[0].astype(jnp.bfloat16)
    wu = wu_ref[0].astype(jnp.bfloat16)
    wd = wd_ref[0].astype(jnp.bfloat16)
    g = lax.dot_general(xb, wg, (((1,), (1,)), ((), ())),
                        preferred_element_type=jnp.float32)
    u = lax.dot_general(xb, wu, (((1,), (1,)), ((), ())),
                        preferred_element_type=jnp.float32)
    h = (g * jax.nn.sigmoid(g) * u).astype(jnp.bfloat16)
    y = lax.dot_general(h, wd, (((1,), (1,)), ((), ())),
                        preferred_element_type=jnp.float32)
    y_ref[...] = y * ws_ref[:, 0:1]


def _moe(block_expert, xs, ws, w_gate, w_up, w_down):
    grid_spec = pltpu.PrefetchScalarGridSpec(
        num_scalar_prefetch=1,
        grid=(NB,),
        in_specs=[
            pl.BlockSpec((BLK, D), lambda i, be: (i, 0)),
            pl.BlockSpec((BLK, 128), lambda i, be: (i, 0)),
            pl.BlockSpec((1, F, D), lambda i, be: (be[0, i], 0, 0)),
            pl.BlockSpec((1, F, D), lambda i, be: (be[0, i], 0, 0)),
            pl.BlockSpec((1, D, F), lambda i, be: (be[0, i], 0, 0)),
        ],
        out_specs=pl.BlockSpec((BLK, D), lambda i, be: (i, 0)),
    )
    return pl.pallas_call(
        _moe_body,
        grid_spec=grid_spec,
        out_shape=jax.ShapeDtypeStruct((NROWS, D), jnp.float32),
    )(block_expert, xs, ws, w_gate, w_up, w_down)


def _shared_body(x_ref, sg_ref, su_ref, sd_ref, o_ref):
    xb = x_ref[...]
    g = lax.dot_general(xb, sg_ref[...], (((1,), (1,)), ((), ())),
                        preferred_element_type=jnp.float32)
    u = lax.dot_general(xb, su_ref[...], (((1,), (1,)), ((), ())),
                        preferred_element_type=jnp.float32)
    h = (g * jax.nn.sigmoid(g) * u).astype(jnp.bfloat16)
    o_ref[...] = lax.dot_general(h, sd_ref[...], (((1,), (1,)), ((), ())),
                                 preferred_element_type=jnp.float32)


def _shared(x_bf, sg_bf, su_bf, sd_bf):
    ntb = T // TB
    return pl.pallas_call(
        _shared_body,
        grid=(ntb,),
        in_specs=[
            pl.BlockSpec((TB, D), lambda i: (i, 0)),
            pl.BlockSpec((FS, D), lambda i: (0, 0)),
            pl.BlockSpec((FS, D), lambda i: (0, 0)),
            pl.BlockSpec((D, FS), lambda i: (0, 0)),
        ],
        out_specs=pl.BlockSpec((TB, D), lambda i: (i, 0)),
        out_shape=jax.ShapeDtypeStruct((T, D), jnp.float32),
    )(x_bf, sg_bf, su_bf, sd_bf)


_NW = 32
_DCH = 32


def _dispatch(x, tok, pos, w_rows):
    mesh = plsc.VectorSubcoreMesh(core_axis_name="c", subcore_axis_name="s")
    per_w = T * K // _NW
    nch = per_w // _DCH

    @functools.partial(
        pl.kernel,
        out_type=(jax.ShapeDtypeStruct((NROWS, D), jnp.float32),
                  jax.ShapeDtypeStruct((NROWS, 128), jnp.float32)),
        mesh=mesh,
        scratch_types=[
            pltpu.VMEM((_DCH,), jnp.int32),
            pltpu.VMEM((_DCH,), jnp.int32),
            pltpu.VMEM((_DCH, D), jnp.float32),
            pltpu.VMEM((_DCH, 128), jnp.float32),
        ],
    )
    def disp(x_hbm, tok_hbm, pos_hbm, wr_hbm, xs_hbm, ws_hbm,
             tokbuf, posbuf, xbuf, wbuf):
        wid = lax.axis_index("s") * 2 + lax.axis_index("c")
        base = wid * per_w
        for c in range(nch):
            off = base + c * _DCH
            pltpu.sync_copy(tok_hbm.at[pl.ds(off, _DCH)], tokbuf)
            pltpu.sync_copy(pos_hbm.at[pl.ds(off, _DCH)], posbuf)
            pltpu.sync_copy(x_hbm.at[tokbuf], xbuf)
            pltpu.sync_copy(xbuf, xs_hbm.at[posbuf])
            pltpu.sync_copy(wr_hbm.at[pl.ds(off, _DCH)], wbuf)
            pltpu.sync_copy(wbuf, ws_hbm.at[posbuf])

    return disp(x, tok, pos, w_rows)


_CCH = 16


def _combine(ys, sh, pos0, pos1):
    mesh = plsc.VectorSubcoreMesh(core_axis_name="c", subcore_axis_name="s")
    per_w = T // _NW
    nch = per_w // _CCH

    @functools.partial(
        pl.kernel,
        out_type=jax.ShapeDtypeStruct((T, D), jnp.float32),
        mesh=mesh,
        scratch_types=[
            pltpu.VMEM((_CCH,), jnp.int32),
            pltpu.VMEM((_CCH,), jnp.int32),
            pltpu.VMEM((_CCH, D), jnp.float32),
            pltpu.VMEM((_CCH, D), jnp.float32),
            pltpu.VMEM((_CCH, D), jnp.float32),
        ],
    )
    def comb(ys_hbm, sh_hbm, p0_hbm, p1_hbm, o_hbm, i0buf, i1buf,
             y0buf, y1buf, obuf):
        wid = lax.axis_index("s") * 2 + lax.axis_index("c")
        base = wid * per_w
        for c in range(nch):
            off = base + c * _CCH
            pltpu.sync_copy(p0_hbm.at[pl.ds(off, _CCH)], i0buf)
            pltpu.sync_copy(p1_hbm.at[pl.ds(off, _CCH)], i1buf)
            pltpu.sync_copy(ys_hbm.at[i0buf], y0buf)
            pltpu.sync_copy(ys_hbm.at[i1buf], y1buf)
            pltpu.sync_copy(sh_hbm.at[pl.ds(off, _CCH)], obuf)

            @pl.loop(0, _CCH)
            def _(r):
                @pl.loop(0, D, step=16)
                def _(j):
                    slc = (r, pl.ds(j, 16))
                    obuf[slc] = obuf[slc] + y0buf[slc] + y1buf[slc]

            pltpu.sync_copy(obuf, o_hbm.at[pl.ds(off, _CCH)])

    return comb(ys, sh, pos0, pos1)


def kernel(x, router_w, w_gate, w_up, w_down, shared_gate, shared_up,
           shared_down):
    w, pos, block_expert = _router(x, router_w)

    tok = jnp.arange(T * K, dtype=jnp.int32) // K
    pos_flat = pos.reshape(T * K)
    w_rows = jnp.broadcast_to(w.reshape(-1)[:, None], (T * K, 128))
    xs, ws = _dispatch(x, tok, pos_flat, w_rows)

    ys = _moe(block_expert, xs, ws, w_gate, w_up, w_down)
    sh = _shared(x.astype(jnp.bfloat16), shared_gate.astype(jnp.bfloat16),
                 shared_up.astype(jnp.bfloat16), shared_down.astype(jnp.bfloat16))

    return _combine(ys, sh, pos[:, 0], pos[:, 1])


# speedup vs baseline: 1.1133x; 1.0526x over previous
import functools

import jax
import jax.numpy as jnp
from jax import lax
from jax.experimental import pallas as pl
from jax.experimental.pallas import tpu as pltpu
from jax.experimental.pallas import tpu_sc as plsc

T, D, E, K, F, FS = 2048, 2048, 8, 2, 1024, 4096
BLK = 128
NB = (T * K) // BLK + E
NROWS = NB * BLK
TB = 128


def _router_body(x_ref, rw_ref, w_ref, pos_ref, be_ref, w0r_ref, w1r_ref):
    xf = x_ref[...]
    rw = rw_ref[...]
    logits = lax.dot_general(xf, rw, (((1,), (1,)), ((), ())),
                             preferred_element_type=jnp.float32)
    m = jnp.max(logits, axis=1, keepdims=True)
    p = jnp.exp(logits - m)
    probs = p / jnp.sum(p, axis=1, keepdims=True)

    eids = lax.broadcasted_iota(jnp.int32, (T, E), 1)
    v1 = jnp.max(probs, axis=1, keepdims=True)
    i1 = jnp.min(jnp.where(probs == v1, eids, E), axis=1, keepdims=True)
    probs2 = jnp.where(eids == i1, -1.0, probs)
    v2 = jnp.max(probs2, axis=1, keepdims=True)
    i2 = jnp.min(jnp.where(probs2 == v2, eids, E), axis=1, keepdims=True)
    s = v1 + v2
    w1 = v1 / s
    w2 = v2 / s

    ind = jnp.where(eids == i1, 1.0, 0.0) + jnp.where(eids == i2, 1.0, 0.0)

    c = ind
    shift = 1
    while shift < T:
        c = c + jnp.concatenate(
            [jnp.zeros((shift, E), jnp.float32), c[: T - shift, :]], axis=0)
        shift *= 2
    c_excl = c - ind
    totals = c[T - 1:T, :]
    nb_e = jnp.floor((totals + (BLK - 1)) / BLK)

    starts = []
    ends = []
    acc = jnp.zeros((1, 1), jnp.float32)
    for e in range(E):
        starts.append(acc)
        acc = acc + nb_e[:, e:e + 1]
        ends.append(acc)

    pos1 = jnp.zeros((T, 1), jnp.float32)
    pos2 = jnp.zeros((T, 1), jnp.float32)
    for e in range(E):
        base = starts[e] * BLK
        pos1 = pos1 + jnp.where(i1 == e, base + c_excl[:, e:e + 1], 0.0)
        pos2 = pos2 + jnp.where(i2 == e, base + c_excl[:, e:e + 1], 0.0)

    b_iota = lax.broadcasted_iota(jnp.int32, (1, NB), 1)
    be = jnp.zeros((1, NB), jnp.int32)
    for e in range(E):
        be = be + jnp.where(b_iota >= ends[e].astype(jnp.int32), 1, 0)
    be_ref[...] = jnp.minimum(be, E - 1)

    w_ref[...] = jnp.concatenate([w1, w2], axis=1)
    pos_ref[...] = jnp.concatenate([pos1, pos2], axis=1).astype(jnp.int32)
    w0r_ref[...] = jnp.broadcast_to(w1, (T, 128))
    w1r_ref[...] = jnp.broadcast_to(w2, (T, 128))


def _router(x, router_w):
    return pl.pallas_call(
        _router_body,
        out_shape=(
            jax.ShapeDtypeStruct((T, K), jnp.float32),
            jax.ShapeDtypeStruct((T, K), jnp.int32),
            jax.ShapeDtypeStruct((1, NB), jnp.int32),
            jax.ShapeDtypeStruct((T, 128), jnp.float32),
            jax.ShapeDtypeStruct((T, 128), jnp.float32),
        ),
    )(x, router_w)


def _moe_body(be_ref, xs_ref, ws_ref, wg_ref, wu_ref, wd_ref, y_ref):
    xb = xs_ref[...].astype(jnp.bfloat16)
    wg = wg_ref[0].astype(jnp.bfloat16)
    wu = wu_ref[0].astype(jnp.bfloat16)
    wd = wd_ref[0].astype(jnp.bfloat16)
    g = lax.dot_general(xb, wg, (((1,), (1,)), ((), ())),
                        preferred_element_type=jnp.float32)
    u = lax.dot_general(xb, wu, (((1,), (1,)), ((), ())),
                        preferred_element_type=jnp.float32)
    h = (g * jax.nn.sigmoid(g) * u).astype(jnp.bfloat16)
    y = lax.dot_general(h, wd, (((1,), (1,)), ((), ())),
                        preferred_element_type=jnp.float32)
    y_ref[...] = y * ws_ref[:, 0:1]


def _moe(block_expert, xs, ws, w_gate, w_up, w_down):
    grid_spec = pltpu.PrefetchScalarGridSpec(
        num_scalar_prefetch=1,
        grid=(NB,),
        in_specs=[
            pl.BlockSpec((BLK, D), lambda i, be: (i, 0)),
            pl.BlockSpec((BLK, 128), lambda i, be: (i, 0)),
            pl.BlockSpec((1, F, D), lambda i, be: (be[0, i], 0, 0)),
            pl.BlockSpec((1, F, D), lambda i, be: (be[0, i], 0, 0)),
            pl.BlockSpec((1, D, F), lambda i, be: (be[0, i], 0, 0)),
        ],
        out_specs=pl.BlockSpec((BLK, D), lambda i, be: (i, 0)),
    )
    return pl.pallas_call(
        _moe_body,
        grid_spec=grid_spec,
        out_shape=jax.ShapeDtypeStruct((NROWS, D), jnp.float32),
    )(block_expert, xs, ws, w_gate, w_up, w_down)


def _shared_body(x_ref, sg_ref, su_ref, sd_ref, o_ref):
    xb = x_ref[...].astype(jnp.bfloat16)
    g = lax.dot_general(xb, sg_ref[...], (((1,), (1,)), ((), ())),
                        preferred_element_type=jnp.float32)
    u = lax.dot_general(xb, su_ref[...], (((1,), (1,)), ((), ())),
                        preferred_element_type=jnp.float32)
    h = (g * jax.nn.sigmoid(g) * u).astype(jnp.bfloat16)
    o_ref[...] = lax.dot_general(h, sd_ref[...], (((1,), (1,)), ((), ())),
                                 preferred_element_type=jnp.float32)


def _shared(x, sg_bf, su_bf, sd_bf):
    ntb = T // TB
    return pl.pallas_call(
        _shared_body,
        grid=(ntb,),
        in_specs=[
            pl.BlockSpec((TB, D), lambda i: (i, 0)),
            pl.BlockSpec((FS, D), lambda i: (0, 0)),
            pl.BlockSpec((FS, D), lambda i: (0, 0)),
            pl.BlockSpec((D, FS), lambda i: (0, 0)),
        ],
        out_specs=pl.BlockSpec((TB, D), lambda i: (i, 0)),
        out_shape=jax.ShapeDtypeStruct((T, D), jnp.float32),
    )(x, sg_bf, su_bf, sd_bf)


_NW = 32
_DCH = 32


def _dispatch(x, pos0_2d, pos1_2d, w0_rows, w1_rows):
    mesh = plsc.VectorSubcoreMesh(core_axis_name="c", subcore_axis_name="s")
    per_w = T // _NW
    nch = per_w // _DCH

    @functools.partial(
        pl.kernel,
        out_type=(jax.ShapeDtypeStruct((NROWS, D), jnp.float32),
                  jax.ShapeDtypeStruct((NROWS, 128), jnp.float32)),
        mesh=mesh,
        scratch_types=[
            pltpu.VMEM((nch, _DCH), jnp.int32),
            pltpu.VMEM((nch, _DCH), jnp.int32),
            pltpu.VMEM((_DCH, D), jnp.float32),
            pltpu.VMEM((_DCH, 128), jnp.float32),
            pltpu.VMEM((_DCH, 128), jnp.float32),
        ],
    )
    def disp(x_hbm, p0_hbm, p1_hbm, w0_hbm, w1_hbm, xs_hbm, ws_hbm,
             p0buf, p1buf, xbuf, w0buf, w1buf):
        wid = lax.axis_index("s") * 2 + lax.axis_index("c")
        pltpu.sync_copy(p0_hbm.at[pl.ds(wid * nch, nch)], p0buf)
        pltpu.sync_copy(p1_hbm.at[pl.ds(wid * nch, nch)], p1buf)
        base = wid * per_w
        for c in range(nch):
            off = base + c * _DCH
            pltpu.sync_copy(x_hbm.at[pl.ds(off, _DCH)], xbuf)
            pltpu.sync_copy(w0_hbm.at[pl.ds(off, _DCH)], w0buf)
            pltpu.sync_copy(w1_hbm.at[pl.ds(off, _DCH)], w1buf)
            pltpu.sync_copy(xbuf, xs_hbm.at[p0buf.at[c]])
            pltpu.sync_copy(xbuf, xs_hbm.at[p1buf.at[c]])
            pltpu.sync_copy(w0buf, ws_hbm.at[p0buf.at[c]])
            pltpu.sync_copy(w1buf, ws_hbm.at[p1buf.at[c]])

    return disp(x, pos0_2d, pos1_2d, w0_rows, w1_rows)


_CCH = 8


def _combine(ys, sh, pos0, pos1):
    mesh = plsc.VectorSubcoreMesh(core_axis_name="c", subcore_axis_name="s")
    per_w = T // _NW
    nch = per_w // _CCH

    @functools.partial(
        pl.kernel,
        out_type=jax.ShapeDtypeStruct((T, D), jnp.float32),
        mesh=mesh,
        scratch_types=[
            pltpu.VMEM((per_w,), jnp.int32),
            pltpu.VMEM((per_w,), jnp.int32),
            [pltpu.VMEM((_CCH, D), jnp.float32) for _ in range(2)],
            [pltpu.VMEM((_CCH, D), jnp.float32) for _ in range(2)],
            [pltpu.VMEM((_CCH, D), jnp.float32) for _ in range(2)],
            [pltpu.SemaphoreType.DMA for _ in range(2)],
            pltpu.SemaphoreType.DMA,
        ],
    )
    def comb(ys_hbm, sh_hbm, p0_hbm, p1_hbm, o_hbm, i0buf, i1buf,
             y0bufs, y1bufs, obufs, gsems, osem):
        wid = lax.axis_index("s") * 2 + lax.axis_index("c")
        base = wid * per_w
        pltpu.sync_copy(p0_hbm.at[pl.ds(base, per_w)], i0buf)
        pltpu.sync_copy(p1_hbm.at[pl.ds(base, per_w)], i1buf)

        pend = {}
        owrite = {}

        def start(c, b):
            off = base + c * _CCH
            ii = pl.ds(c * _CCH, _CCH)
            pend[b] = (
                pltpu.async_copy(ys_hbm.at[i0buf.at[ii]], y0bufs[b],
                                 gsems[b]),
                pltpu.async_copy(ys_hbm.at[i1buf.at[ii]], y1bufs[b],
                                 gsems[b]),
                pltpu.async_copy(sh_hbm.at[pl.ds(off, _CCH)], obufs[b],
                                 gsems[b]),
            )

        start(0, 0)
        for c in range(nch):
            b = c % 2
            if c + 1 < nch:
                if (1 - b) in owrite:
                    owrite.pop(1 - b).wait()
                start(c + 1, 1 - b)
            for d in pend.pop(b):
                d.wait()
            y0buf, y1buf, obuf = y0bufs[b], y1bufs[b], obufs[b]

            @pl.loop(0, _CCH)
            def _(r):
                @pl.loop(0, D, step=16)
                def _(j):
                    slc = (r, pl.ds(j, 16))
                    obuf[slc] = obuf[slc] + y0buf[slc] + y1buf[slc]

            owrite[b] = pltpu.async_copy(
                obuf, o_hbm.at[pl.ds(base + c * _CCH, _CCH)], osem)
        for d in owrite.values():
            d.wait()

    return comb(ys, sh, pos0, pos1)


def kernel(x, router_w, w_gate, w_up, w_down, shared_gate, shared_up,
           shared_down):
    w, pos, block_expert, w0_rows, w1_rows = _router(x, router_w)

    pos0 = pos[:, 0]
    pos1 = pos[:, 1]
    xs, ws = _dispatch(x, pos0.reshape(T // _DCH, _DCH),
                       pos1.reshape(T // _DCH, _DCH), w0_rows, w1_rows)

    ys = _moe(block_expert, xs, ws, w_gate, w_up, w_down)
    sh = _shared(x, shared_gate.astype(jnp.bfloat16),
                 shared_up.astype(jnp.bfloat16),
                 shared_down.astype(jnp.bfloat16))

    return _combine(ys, sh, pos0, pos1)


# speedup vs baseline: 1.1192x; 1.0054x over previous
import functools

import jax
import jax.numpy as jnp
from jax import lax
from jax.experimental import pallas as pl
from jax.experimental.pallas import tpu as pltpu
from jax.experimental.pallas import tpu_sc as plsc

T, D, E, K, F, FS = 2048, 2048, 8, 2, 1024, 4096
BLK = 128
NB = (T * K) // BLK + E
NROWS = NB * BLK
TB = 128


def _router_body(x_ref, rw_ref, w_ref, pos_ref, be_ref, w0r_ref, w1r_ref):
    xf = x_ref[...]
    rw = rw_ref[...]
    logits = lax.dot_general(xf, rw, (((1,), (1,)), ((), ())),
                             preferred_element_type=jnp.float32)
    m = jnp.max(logits, axis=1, keepdims=True)
    p = jnp.exp(logits - m)
    probs = p / jnp.sum(p, axis=1, keepdims=True)

    eids = lax.broadcasted_iota(jnp.int32, (T, E), 1)
    v1 = jnp.max(probs, axis=1, keepdims=True)
    i1 = jnp.min(jnp.where(probs == v1, eids, E), axis=1, keepdims=True)
    probs2 = jnp.where(eids == i1, -1.0, probs)
    v2 = jnp.max(probs2, axis=1, keepdims=True)
    i2 = jnp.min(jnp.where(probs2 == v2, eids, E), axis=1, keepdims=True)
    s = v1 + v2
    w1 = v1 / s
    w2 = v2 / s

    ind = jnp.where(eids == i1, 1.0, 0.0) + jnp.where(eids == i2, 1.0, 0.0)

    c = ind
    shift = 1
    while shift < T:
        c = c + jnp.concatenate(
            [jnp.zeros((shift, E), jnp.float32), c[: T - shift, :]], axis=0)
        shift *= 2
    c_excl = c - ind
    totals = c[T - 1:T, :]
    nb_e = jnp.floor((totals + (BLK - 1)) / BLK)

    starts = []
    ends = []
    acc = jnp.zeros((1, 1), jnp.float32)
    for e in range(E):
        starts.append(acc)
        acc = acc + nb_e[:, e:e + 1]
        ends.append(acc)

    pos1 = jnp.zeros((T, 1), jnp.float32)
    pos2 = jnp.zeros((T, 1), jnp.float32)
    for e in range(E):
        base = starts[e] * BLK
        pos1 = pos1 + jnp.where(i1 == e, base + c_excl[:, e:e + 1], 0.0)
        pos2 = pos2 + jnp.where(i2 == e, base + c_excl[:, e:e + 1], 0.0)

    b_iota = lax.broadcasted_iota(jnp.int32, (1, NB), 1)
    be = jnp.zeros((1, NB), jnp.int32)
    for e in range(E):
        be = be + jnp.where(b_iota >= ends[e].astype(jnp.int32), 1, 0)
    be_ref[...] = jnp.minimum(be, E - 1)

    w_ref[...] = jnp.concatenate([w1, w2], axis=1)
    pos_ref[...] = jnp.concatenate([pos1, pos2], axis=1).astype(jnp.int32)
    w0r_ref[...] = jnp.broadcast_to(w1, (T, 128))
    w1r_ref[...] = jnp.broadcast_to(w2, (T, 128))


def _router(x, router_w):
    return pl.pallas_call(
        _router_body,
        out_shape=(
            jax.ShapeDtypeStruct((T, K), jnp.float32),
            jax.ShapeDtypeStruct((T, K), jnp.int32),
            jax.ShapeDtypeStruct((1, NB), jnp.int32),
            jax.ShapeDtypeStruct((T, 128), jnp.float32),
            jax.ShapeDtypeStruct((T, 128), jnp.float32),
        ),
    )(x, router_w)


def _moe_body(be_ref, xs_ref, ws_ref, wg_hbm, wu_hbm, wd_hbm, y_ref,
              wgbuf, wubuf, wdbuf, sems):
    i = pl.program_id(0)
    e = be_ref[0, i]
    slot = lax.rem(e, 2)

    def issue(ex, sl):
        pltpu.async_copy(wg_hbm.at[ex], wgbuf.at[sl], sems.at[sl])
        pltpu.async_copy(wu_hbm.at[ex], wubuf.at[sl], sems.at[sl])
        pltpu.async_copy(wd_hbm.at[ex], wdbuf.at[sl], sems.at[sl])

    def drain(ex, sl):
        pltpu.make_async_copy(wg_hbm.at[ex], wgbuf.at[sl], sems.at[sl]).wait()
        pltpu.make_async_copy(wu_hbm.at[ex], wubuf.at[sl], sems.at[sl]).wait()
        pltpu.make_async_copy(wd_hbm.at[ex], wdbuf.at[sl], sems.at[sl]).wait()

    @pl.when(i == 0)
    def _():
        issue(e, slot)

    prev = be_ref[0, jnp.maximum(i - 1, 0)]

    @pl.when(jnp.logical_or(i == 0, prev != e))
    def _():
        drain(e, slot)

    nxt = be_ref[0, jnp.minimum(i + 1, NB - 1)]

    @pl.when(jnp.logical_and(i + 1 < NB, nxt != e))
    def _():
        issue(nxt, lax.rem(nxt, 2))

    xb = xs_ref[...].astype(jnp.bfloat16)
    wg = wgbuf[slot].astype(jnp.bfloat16)
    wu = wubuf[slot].astype(jnp.bfloat16)
    wd = wdbuf[slot].astype(jnp.bfloat16)
    g = lax.dot_general(xb, wg, (((1,), (1,)), ((), ())),
                        preferred_element_type=jnp.float32)
    u = lax.dot_general(xb, wu, (((1,), (1,)), ((), ())),
                        preferred_element_type=jnp.float32)
    h = (g * jax.nn.sigmoid(g) * u).astype(jnp.bfloat16)
    y = lax.dot_general(h, wd, (((1,), (1,)), ((), ())),
                        preferred_element_type=jnp.float32)
    y_ref[...] = y * ws_ref[:, 0:1]


def _moe(block_expert, xs, ws, w_gate, w_up, w_down):
    grid_spec = pltpu.PrefetchScalarGridSpec(
        num_scalar_prefetch=1,
        grid=(NB,),
        in_specs=[
            pl.BlockSpec((BLK, D), lambda i, be: (i, 0)),
            pl.BlockSpec((BLK, 128), lambda i, be: (i, 0)),
            pl.BlockSpec(memory_space=pl.ANY),
            pl.BlockSpec(memory_space=pl.ANY),
            pl.BlockSpec(memory_space=pl.ANY),
        ],
        out_specs=pl.BlockSpec((BLK, D), lambda i, be: (i, 0)),
        scratch_shapes=[
            pltpu.VMEM((2, F, D), jnp.float32),
            pltpu.VMEM((2, F, D), jnp.float32),
            pltpu.VMEM((2, D, F), jnp.float32),
            pltpu.SemaphoreType.DMA((2,)),
        ],
    )
    return pl.pallas_call(
        _moe_body,
        grid_spec=grid_spec,
        out_shape=jax.ShapeDtypeStruct((NROWS, D), jnp.float32),
    )(block_expert, xs, ws, w_gate, w_up, w_down)


def _shared_body(x_ref, sg_hbm, su_hbm, sd_hbm, o_ref, sgbuf, subuf, sdbuf,
                 sem):
    i = pl.program_id(0)

    @pl.when(i == 0)
    def _():
        pltpu.async_copy(sg_hbm, sgbuf, sem)
        pltpu.async_copy(su_hbm, subuf, sem)
        pltpu.async_copy(sd_hbm, sdbuf, sem)
        pltpu.make_async_copy(sg_hbm, sgbuf, sem).wait()
        pltpu.make_async_copy(su_hbm, subuf, sem).wait()
        pltpu.make_async_copy(sd_hbm, sdbuf, sem).wait()

    xb = x_ref[...].astype(jnp.bfloat16)
    g = lax.dot_general(xb, sgbuf[...], (((1,), (1,)), ((), ())),
                        preferred_element_type=jnp.float32)
    u = lax.dot_general(xb, subuf[...], (((1,), (1,)), ((), ())),
                        preferred_element_type=jnp.float32)
    h = (g * jax.nn.sigmoid(g) * u).astype(jnp.bfloat16)
    o_ref[...] = lax.dot_general(h, sdbuf[...], (((1,), (1,)), ((), ())),
                                 preferred_element_type=jnp.float32)


def _shared(x, sg_bf, su_bf, sd_bf):
    ntb = T // TB
    return pl.pallas_call(
        _shared_body,
        grid=(ntb,),
        in_specs=[
            pl.BlockSpec((TB, D), lambda i: (i, 0)),
            pl.BlockSpec(memory_space=pl.ANY),
            pl.BlockSpec(memory_space=pl.ANY),
            pl.BlockSpec(memory_space=pl.ANY),
        ],
        out_specs=pl.BlockSpec((TB, D), lambda i: (i, 0)),
        out_shape=jax.ShapeDtypeStruct((T, D), jnp.float32),
        scratch_shapes=[
            pltpu.VMEM((FS, D), jnp.bfloat16),
            pltpu.VMEM((FS, D), jnp.bfloat16),
            pltpu.VMEM((D, FS), jnp.bfloat16),
            pltpu.SemaphoreType.DMA,
        ],
    )(x, sg_bf, su_bf, sd_bf)


_NW = 32
_DCH = 32


def _dispatch(x, pos0_2d, pos1_2d, w0_rows, w1_rows):
    mesh = plsc.VectorSubcoreMesh(core_axis_name="c", subcore_axis_name="s")
    per_w = T // _NW
    nch = per_w // _DCH

    @functools.partial(
        pl.kernel,
        out_type=(jax.ShapeDtypeStruct((NROWS, D), jnp.float32),
                  jax.ShapeDtypeStruct((NROWS, 128), jnp.float32)),
        mesh=mesh,
        scratch_types=[
            pltpu.VMEM((nch, _DCH), jnp.int32),
            pltpu.VMEM((nch, _DCH), jnp.int32),
            pltpu.VMEM((_DCH, D), jnp.float32),
            pltpu.VMEM((_DCH, 128), jnp.float32),
            pltpu.VMEM((_DCH, 128), jnp.float32),
        ],
    )
    def disp(x_hbm, p0_hbm, p1_hbm, w0_hbm, w1_hbm, xs_hbm, ws_hbm,
             p0buf, p1buf, xbuf, w0buf, w1buf):
        wid = lax.axis_index("s") * 2 + lax.axis_index("c")
        pltpu.sync_copy(p0_hbm.at[pl.ds(wid * nch, nch)], p0buf)
        pltpu.sync_copy(p1_hbm.at[pl.ds(wid * nch, nch)], p1buf)
        base = wid * per_w
        for c in range(nch):
            off = base + c * _DCH
            pltpu.sync_copy(x_hbm.at[pl.ds(off, _DCH)], xbuf)
            pltpu.sync_copy(w0_hbm.at[pl.ds(off, _DCH)], w0buf)
            pltpu.sync_copy(w1_hbm.at[pl.ds(off, _DCH)], w1buf)
            pltpu.sync_copy(xbuf, xs_hbm.at[p0buf.at[c]])
            pltpu.sync_copy(xbuf, xs_hbm.at[p1buf.at[c]])
            pltpu.sync_copy(w0buf, ws_hbm.at[p0buf.at[c]])
            pltpu.sync_copy(w1buf, ws_hbm.at[p1buf.at[c]])

    return disp(x, pos0_2d, pos1_2d, w0_rows, w1_rows)


_CCH = 8


def _combine(ys, sh, pos0, pos1):
    mesh = plsc.VectorSubcoreMesh(core_axis_name="c", subcore_axis_name="s")
    per_w = T // _NW
    nch = per_w // _CCH

    @functools.partial(
        pl.kernel,
        out_type=jax.ShapeDtypeStruct((T, D), jnp.float32),
        mesh=mesh,
        scratch_types=[
            pltpu.VMEM((per_w,), jnp.int32),
            pltpu.VMEM((per_w,), jnp.int32),
            [pltpu.VMEM((_CCH, D), jnp.float32) for _ in range(2)],
            [pltpu.VMEM((_CCH, D), jnp.float32) for _ in range(2)],
            [pltpu.VMEM((_CCH, D), jnp.float32) for _ in range(2)],
            [pltpu.SemaphoreType.DMA for _ in range(2)],
            pltpu.SemaphoreType.DMA,
        ],
    )
    def comb(ys_hbm, sh_hbm, p0_hbm, p1_hbm, o_hbm, i0buf, i1buf,
             y0bufs, y1bufs, obufs, gsems, osem):
        wid = lax.axis_index("s") * 2 + lax.axis_index("c")
        base = wid * per_w
        pltpu.sync_copy(p0_hbm.at[pl.ds(base, per_w)], i0buf)
        pltpu.sync_copy(p1_hbm.at[pl.ds(base, per_w)], i1buf)

        pend = {}
        owrite = {}

        def start(c, b):
            off = base + c * _CCH
            ii = pl.ds(c * _CCH, _CCH)
            pend[b] = (
                pltpu.async_copy(ys_hbm.at[i0buf.at[ii]], y0bufs[b],
                                 gsems[b]),
                pltpu.async_copy(ys_hbm.at[i1buf.at[ii]], y1bufs[b],
                                 gsems[b]),
                pltpu.async_copy(sh_hbm.at[pl.ds(off, _CCH)], obufs[b],
                                 gsems[b]),
            )

        start(0, 0)
        for c in range(nch):
            b = c % 2
            if c + 1 < nch:
                if (1 - b) in owrite:
                    owrite.pop(1 - b).wait()
                start(c + 1, 1 - b)
            for d in pend.pop(b):
                d.wait()
            y0buf, y1buf, obuf = y0bufs[b], y1bufs[b], obufs[b]

            @pl.loop(0, _CCH)
            def _(r):
                @pl.loop(0, D, step=16)
                def _(j):
                    slc = (r, pl.ds(j, 16))
                    obuf[slc] = obuf[slc] + y0buf[slc] + y1buf[slc]

            owrite[b] = pltpu.async_copy(
                obuf, o_hbm.at[pl.ds(base + c * _CCH, _CCH)], osem)
        for d in owrite.values():
            d.wait()

    return comb(ys, sh, pos0, pos1)


def kernel(x, router_w, w_gate, w_up, w_down, shared_gate, shared_up,
           shared_down):
    w, pos, block_expert, w0_rows, w1_rows = _router(x, router_w)

    pos0 = pos[:, 0]
    pos1 = pos[:, 1]
    xs, ws = _dispatch(x, pos0.reshape(T // _DCH, _DCH),
                       pos1.reshape(T // _DCH, _DCH), w0_rows, w1_rows)

    ys = _moe(block_expert, xs, ws, w_gate, w_up, w_down)
    sh = _shared(x, shared_gate.astype(jnp.bfloat16),
                 shared_up.astype(jnp.bfloat16),
                 shared_down.astype(jnp.bfloat16))

    return _combine(ys, sh, pos0, pos1)


# speedup vs baseline: 1.8586x; 1.6606x over previous
import functools

import jax
import jax.numpy as jnp
from jax import lax
from jax.experimental import pallas as pl
from jax.experimental.pallas import tpu as pltpu
from jax.experimental.pallas import tpu_sc as plsc

T, D, E, K, F, FS = 2048, 2048, 8, 2, 1024, 4096
BLK = 256
NB = (T * K) // BLK + E
NROWS = NB * BLK
FSB = 256


def _router_body(x_ref, rw_ref, w_ref, pos_ref, be_ref, w0r_ref, w1r_ref):
    xf = x_ref[...]
    rw = rw_ref[...]
    logits = lax.dot_general(xf, rw, (((1,), (1,)), ((), ())),
                             preferred_element_type=jnp.float32)
    m = jnp.max(logits, axis=1, keepdims=True)
    p = jnp.exp(logits - m)
    probs = p / jnp.sum(p, axis=1, keepdims=True)

    eids = lax.broadcasted_iota(jnp.int32, (T, E), 1)
    v1 = jnp.max(probs, axis=1, keepdims=True)
    i1 = jnp.min(jnp.where(probs == v1, eids, E), axis=1, keepdims=True)
    probs2 = jnp.where(eids == i1, -1.0, probs)
    v2 = jnp.max(probs2, axis=1, keepdims=True)
    i2 = jnp.min(jnp.where(probs2 == v2, eids, E), axis=1, keepdims=True)
    s = v1 + v2
    w1 = v1 / s
    w2 = v2 / s

    ind = jnp.where(eids == i1, 1.0, 0.0) + jnp.where(eids == i2, 1.0, 0.0)

    c = ind
    shift = 1
    while shift < T:
        c = c + jnp.concatenate(
            [jnp.zeros((shift, E), jnp.float32), c[: T - shift, :]], axis=0)
        shift *= 2
    c_excl = c - ind
    totals = c[T - 1:T, :]
    nb_e = jnp.floor((totals + (BLK - 1)) / BLK)

    starts = []
    ends = []
    acc = jnp.zeros((1, 1), jnp.float32)
    for e in range(E):
        starts.append(acc)
        acc = acc + nb_e[:, e:e + 1]
        ends.append(acc)

    pos1 = jnp.zeros((T, 1), jnp.float32)
    pos2 = jnp.zeros((T, 1), jnp.float32)
    for e in range(E):
        base = starts[e] * BLK
        pos1 = pos1 + jnp.where(i1 == e, base + c_excl[:, e:e + 1], 0.0)
        pos2 = pos2 + jnp.where(i2 == e, base + c_excl[:, e:e + 1], 0.0)

    b_iota = lax.broadcasted_iota(jnp.int32, (1, NB), 1)
    be = jnp.zeros((1, NB), jnp.int32)
    for e in range(E):
        be = be + jnp.where(b_iota >= ends[e].astype(jnp.int32), 1, 0)
    be_ref[...] = jnp.minimum(be, E - 1)

    w_ref[...] = jnp.concatenate([w1, w2], axis=1)
    pos_ref[...] = jnp.concatenate([pos1, pos2], axis=1).astype(jnp.int32)
    w0r_ref[...] = jnp.broadcast_to(w1, (T, 128))
    w1r_ref[...] = jnp.broadcast_to(w2, (T, 128))


def _router(x, router_w):
    return pl.pallas_call(
        _router_body,
        out_shape=(
            jax.ShapeDtypeStruct((T, K), jnp.float32),
            jax.ShapeDtypeStruct((T, K), jnp.int32),
            jax.ShapeDtypeStruct((1, NB), jnp.int32),
            jax.ShapeDtypeStruct((T, 128), jnp.float32),
            jax.ShapeDtypeStruct((T, 128), jnp.float32),
        ),
    )(x, router_w)


def _moe_body(be_ref, xs_ref, ws_ref, wg_hbm, wu_hbm, wd_hbm, y_ref,
              wgbuf, wubuf, wdbuf, sems):
    i = pl.program_id(0)
    e = be_ref[0, i]
    slot = lax.rem(e, 2)

    def issue(ex, sl):
        pltpu.async_copy(wg_hbm.at[ex], wgbuf.at[sl], sems.at[sl])
        pltpu.async_copy(wu_hbm.at[ex], wubuf.at[sl], sems.at[sl])
        pltpu.async_copy(wd_hbm.at[ex], wdbuf.at[sl], sems.at[sl])

    def drain(ex, sl):
        pltpu.make_async_copy(wg_hbm.at[ex], wgbuf.at[sl], sems.at[sl]).wait()
        pltpu.make_async_copy(wu_hbm.at[ex], wubuf.at[sl], sems.at[sl]).wait()
        pltpu.make_async_copy(wd_hbm.at[ex], wdbuf.at[sl], sems.at[sl]).wait()

    @pl.when(i == 0)
    def _():
        issue(e, slot)

    prev = be_ref[0, jnp.maximum(i - 1, 0)]

    @pl.when(jnp.logical_or(i == 0, prev != e))
    def _():
        drain(e, slot)

    nxt = be_ref[0, jnp.minimum(i + 1, NB - 1)]

    @pl.when(jnp.logical_and(i + 1 < NB, nxt != e))
    def _():
        issue(nxt, lax.rem(nxt, 2))

    xb = xs_ref[...].astype(jnp.bfloat16)
    wg = wgbuf[slot].astype(jnp.bfloat16)
    wu = wubuf[slot].astype(jnp.bfloat16)
    wd = wdbuf[slot].astype(jnp.bfloat16)
    g = lax.dot_general(xb, wg, (((1,), (1,)), ((), ())),
                        preferred_element_type=jnp.float32)
    u = lax.dot_general(xb, wu, (((1,), (1,)), ((), ())),
                        preferred_element_type=jnp.float32)
    h = (g * jax.nn.sigmoid(g) * u).astype(jnp.bfloat16)
    y = lax.dot_general(h, wd, (((1,), (1,)), ((), ())),
                        preferred_element_type=jnp.float32)
    y_ref[...] = y * ws_ref[:, 0:1]


def _moe(block_expert, xs, ws, w_gate, w_up, w_down):
    grid_spec = pltpu.PrefetchScalarGridSpec(
        num_scalar_prefetch=1,
        grid=(NB,),
        in_specs=[
            pl.BlockSpec((BLK, D), lambda i, be: (i, 0)),
            pl.BlockSpec((BLK, 128), lambda i, be: (i, 0)),
            pl.BlockSpec(memory_space=pl.ANY),
            pl.BlockSpec(memory_space=pl.ANY),
            pl.BlockSpec(memory_space=pl.ANY),
        ],
        out_specs=pl.BlockSpec((BLK, D), lambda i, be: (i, 0)),
        scratch_shapes=[
            pltpu.VMEM((2, F, D), jnp.float32),
            pltpu.VMEM((2, F, D), jnp.float32),
            pltpu.VMEM((2, D, F), jnp.float32),
            pltpu.SemaphoreType.DMA((2,)),
        ],
    )
    return pl.pallas_call(
        _moe_body,
        grid_spec=grid_spec,
        out_shape=jax.ShapeDtypeStruct((NROWS, D), jnp.float32),
    )(block_expert, xs, ws, w_gate, w_up, w_down)


def _shared_body(x_ref, sg_ref, su_ref, sd_ref, o_ref):
    xb = x_ref[...]
    sg = sg_ref[...].astype(jnp.bfloat16)
    su = su_ref[...].astype(jnp.bfloat16)
    sd = sd_ref[...].astype(jnp.bfloat16)
    g = lax.dot_general(xb, sg, (((1,), (1,)), ((), ())),
                        preferred_element_type=jnp.float32)
    u = lax.dot_general(xb, su, (((1,), (1,)), ((), ())),
                        preferred_element_type=jnp.float32)
    h = (g * jax.nn.sigmoid(g) * u).astype(jnp.bfloat16)
    part = lax.dot_general(h, sd, (((1,), (1,)), ((), ())),
                           preferred_element_type=jnp.float32)

    @pl.when(pl.program_id(0) == 0)
    def _():
        o_ref[...] = part

    @pl.when(pl.program_id(0) != 0)
    def _():
        o_ref[...] += part


def _shared(x_bf, shared_gate, shared_up, shared_down):
    nfb = FS // FSB
    return pl.pallas_call(
        _shared_body,
        grid=(nfb,),
        in_specs=[
            pl.BlockSpec((T, D), lambda i: (0, 0)),
            pl.BlockSpec((FSB, D), lambda i: (i, 0)),
            pl.BlockSpec((FSB, D), lambda i: (i, 0)),
            pl.BlockSpec((D, FSB), lambda i: (0, i)),
        ],
        out_specs=pl.BlockSpec((T, D), lambda i: (0, 0)),
        out_shape=jax.ShapeDtypeStruct((T, D), jnp.float32),
    )(x_bf, shared_gate, shared_up, shared_down)


_NW = 32
_DCH = 32


def _dispatch(x, pos0_2d, pos1_2d, w0_rows, w1_rows):
    mesh = plsc.VectorSubcoreMesh(core_axis_name="c", subcore_axis_name="s")
    per_w = T // _NW
    nch = per_w // _DCH

    @functools.partial(
        pl.kernel,
        out_type=(jax.ShapeDtypeStruct((NROWS, D), jnp.float32),
                  jax.ShapeDtypeStruct((NROWS, 128), jnp.float32)),
        mesh=mesh,
        scratch_types=[
            pltpu.VMEM((nch, _DCH), jnp.int32),
            pltpu.VMEM((nch, _DCH), jnp.int32),
            pltpu.VMEM((_DCH, D), jnp.float32),
            pltpu.VMEM((_DCH, 128), jnp.float32),
            pltpu.VMEM((_DCH, 128), jnp.float32),
        ],
    )
    def disp(x_hbm, p0_hbm, p1_hbm, w0_hbm, w1_hbm, xs_hbm, ws_hbm,
             p0buf, p1buf, xbuf, w0buf, w1buf):
        wid = lax.axis_index("s") * 2 + lax.axis_index("c")
        pltpu.sync_copy(p0_hbm.at[pl.ds(wid * nch, nch)], p0buf)
        pltpu.sync_copy(p1_hbm.at[pl.ds(wid * nch, nch)], p1buf)
        base = wid * per_w
        for c in range(nch):
            off = base + c * _DCH
            pltpu.sync_copy(x_hbm.at[pl.ds(off, _DCH)], xbuf)
            pltpu.sync_copy(w0_hbm.at[pl.ds(off, _DCH)], w0buf)
            pltpu.sync_copy(w1_hbm.at[pl.ds(off, _DCH)], w1buf)
            pltpu.sync_copy(xbuf, xs_hbm.at[p0buf.at[c]])
            pltpu.sync_copy(xbuf, xs_hbm.at[p1buf.at[c]])
            pltpu.sync_copy(w0buf, ws_hbm.at[p0buf.at[c]])
            pltpu.sync_copy(w1buf, ws_hbm.at[p1buf.at[c]])

    return disp(x, pos0_2d, pos1_2d, w0_rows, w1_rows)


_CCH = 8


def _combine(ys, sh, pos0, pos1):
    mesh = plsc.VectorSubcoreMesh(core_axis_name="c", subcore_axis_name="s")
    per_w = T // _NW
    nch = per_w // _CCH

    @functools.partial(
        pl.kernel,
        out_type=jax.ShapeDtypeStruct((T, D), jnp.float32),
        mesh=mesh,
        scratch_types=[
            pltpu.VMEM((per_w,), jnp.int32),
            pltpu.VMEM((per_w,), jnp.int32),
            [pltpu.VMEM((_CCH, D), jnp.float32) for _ in range(2)],
            [pltpu.VMEM((_CCH, D), jnp.float32) for _ in range(2)],
            [pltpu.VMEM((_CCH, D), jnp.float32) for _ in range(2)],
            [pltpu.SemaphoreType.DMA for _ in range(2)],
            pltpu.SemaphoreType.DMA,
        ],
    )
    def comb(ys_hbm, sh_hbm, p0_hbm, p1_hbm, o_hbm, i0buf, i1buf,
             y0bufs, y1bufs, obufs, gsems, osem):
        wid = lax.axis_index("s") * 2 + lax.axis_index("c")
        base = wid * per_w
        pltpu.sync_copy(p0_hbm.at[pl.ds(base, per_w)], i0buf)
        pltpu.sync_copy(p1_hbm.at[pl.ds(base, per_w)], i1buf)

        pend = {}
        owrite = {}

        def start(c, b):
            off = base + c * _CCH
            ii = pl.ds(c * _CCH, _CCH)
            pend[b] = (
                pltpu.async_copy(ys_hbm.at[i0buf.at[ii]], y0bufs[b],
                                 gsems[b]),
                pltpu.async_copy(ys_hbm.at[i1buf.at[ii]], y1bufs[b],
                                 gsems[b]),
                pltpu.async_copy(sh_hbm.at[pl.ds(off, _CCH)], obufs[b],
                                 gsems[b]),
            )

        start(0, 0)
        for c in range(nch):
            b = c % 2
            if c + 1 < nch:
                if (1 - b) in owrite:
                    owrite.pop(1 - b).wait()
                start(c + 1, 1 - b)
            for d in pend.pop(b):
                d.wait()
            y0buf, y1buf, obuf = y0bufs[b], y1bufs[b], obufs[b]

            @pl.loop(0, _CCH)
            def _(r):
                @pl.loop(0, D, step=16)
                def _(j):
                    slc = (r, pl.ds(j, 16))
                    obuf[slc] = obuf[slc] + y0buf[slc] + y1buf[slc]

            owrite[b] = pltpu.async_copy(
                obuf, o_hbm.at[pl.ds(base + c * _CCH, _CCH)], osem)
        for d in owrite.values():
            d.wait()

    return comb(ys, sh, pos0, pos1)


def kernel(x, router_w, w_gate, w_up, w_down, shared_gate, shared_up,
           shared_down):
    w, pos, block_expert, w0_rows, w1_rows = _router(x, router_w)

    pos0 = pos[:, 0]
    pos1 = pos[:, 1]
    xs, ws = _dispatch(x, pos0.reshape(T // _DCH, _DCH),
                       pos1.reshape(T // _DCH, _DCH), w0_rows, w1_rows)

    ys = _moe(block_expert, xs, ws, w_gate, w_up, w_down)
    sh = _shared(x.astype(jnp.bfloat16), shared_gate, shared_up, shared_down)

    return _combine(ys, sh, pos0, pos1)


# speedup vs baseline: 1.8883x; 1.0160x over previous
import functools

import jax
import jax.numpy as jnp
from jax import lax
from jax.experimental import pallas as pl
from jax.experimental.pallas import tpu as pltpu
from jax.experimental.pallas import tpu_sc as plsc

T, D, E, K, F, FS = 2048, 2048, 8, 2, 1024, 4096
BLK = 256
NB = (T * K) // BLK + E
NROWS = NB * BLK
FSB = 256


def _router_body(x_ref, rw_ref, w_ref, pos_ref, be_ref, w0r_ref, w1r_ref):
    xf = x_ref[...]
    rw = rw_ref[...]
    logits = lax.dot_general(xf, rw, (((1,), (1,)), ((), ())),
                             preferred_element_type=jnp.float32)
    m = jnp.max(logits, axis=1, keepdims=True)
    p = jnp.exp(logits - m)
    probs = p / jnp.sum(p, axis=1, keepdims=True)

    eids = lax.broadcasted_iota(jnp.int32, (T, E), 1)
    v1 = jnp.max(probs, axis=1, keepdims=True)
    i1 = jnp.min(jnp.where(probs == v1, eids, E), axis=1, keepdims=True)
    probs2 = jnp.where(eids == i1, -1.0, probs)
    v2 = jnp.max(probs2, axis=1, keepdims=True)
    i2 = jnp.min(jnp.where(probs2 == v2, eids, E), axis=1, keepdims=True)
    s = v1 + v2
    w1 = v1 / s
    w2 = v2 / s

    ind = jnp.where(eids == i1, 1.0, 0.0) + jnp.where(eids == i2, 1.0, 0.0)

    c = ind
    shift = 1
    while shift < T:
        c = c + jnp.concatenate(
            [jnp.zeros((shift, E), jnp.float32), c[: T - shift, :]], axis=0)
        shift *= 2
    c_excl = c - ind
    totals = c[T - 1:T, :]
    nb_e = jnp.floor((totals + (BLK - 1)) / BLK)

    starts = []
    ends = []
    acc = jnp.zeros((1, 1), jnp.float32)
    for e in range(E):
        starts.append(acc)
        acc = acc + nb_e[:, e:e + 1]
        ends.append(acc)

    pos1 = jnp.zeros((T, 1), jnp.float32)
    pos2 = jnp.zeros((T, 1), jnp.float32)
    for e in range(E):
        base = starts[e] * BLK
        pos1 = pos1 + jnp.where(i1 == e, base + c_excl[:, e:e + 1], 0.0)
        pos2 = pos2 + jnp.where(i2 == e, base + c_excl[:, e:e + 1], 0.0)

    b_iota = lax.broadcasted_iota(jnp.int32, (1, NB + 8), 1)
    be = jnp.zeros((1, NB + 8), jnp.int32)
    for e in range(E):
        be = be + jnp.where(b_iota >= ends[e].astype(jnp.int32), 1, 0)
    be = jnp.minimum(be, E - 1)
    used = ends[E - 1].astype(jnp.int32)
    be_ref[...] = jnp.where(b_iota == NB, used, be)

    w_ref[...] = jnp.concatenate([w1, w2], axis=1)
    pos_ref[...] = jnp.concatenate([pos1, pos2], axis=1).astype(jnp.int32)
    w0r_ref[...] = jnp.broadcast_to(w1, (T, 128))
    w1r_ref[...] = jnp.broadcast_to(w2, (T, 128))


def _router(x, router_w):
    return pl.pallas_call(
        _router_body,
        out_shape=(
            jax.ShapeDtypeStruct((T, K), jnp.float32),
            jax.ShapeDtypeStruct((T, K), jnp.int32),
            jax.ShapeDtypeStruct((1, NB + 8), jnp.int32),
            jax.ShapeDtypeStruct((T, 128), jnp.float32),
            jax.ShapeDtypeStruct((T, 128), jnp.float32),
        ),
    )(x, router_w)


def _moe_body(be_ref, xs_ref, ws_ref, wg_hbm, wu_hbm, wd_hbm, y_ref,
              wgbuf, wubuf, wdbuf, sems):
    i = pl.program_id(0)
    e = be_ref[0, i]
    slot = lax.rem(e, 2)

    def issue(ex, sl):
        pltpu.async_copy(wg_hbm.at[ex], wgbuf.at[sl], sems.at[sl])
        pltpu.async_copy(wu_hbm.at[ex], wubuf.at[sl], sems.at[sl])
        pltpu.async_copy(wd_hbm.at[ex], wdbuf.at[sl], sems.at[sl])

    def drain(ex, sl):
        pltpu.make_async_copy(wg_hbm.at[ex], wgbuf.at[sl], sems.at[sl]).wait()
        pltpu.make_async_copy(wu_hbm.at[ex], wubuf.at[sl], sems.at[sl]).wait()
        pltpu.make_async_copy(wd_hbm.at[ex], wdbuf.at[sl], sems.at[sl]).wait()

    @pl.when(i == 0)
    def _():
        issue(e, slot)

    prev = be_ref[0, jnp.maximum(i - 1, 0)]

    @pl.when(jnp.logical_or(i == 0, prev != e))
    def _():
        drain(e, slot)

    nxt = be_ref[0, jnp.minimum(i + 1, NB - 1)]

    @pl.when(jnp.logical_and(i + 1 < NB, nxt != e))
    def _():
        issue(nxt, lax.rem(nxt, 2))

    @pl.when(i < be_ref[0, NB])
    def _():
        xb = xs_ref[...].astype(jnp.bfloat16)
        wg = wgbuf[slot].astype(jnp.bfloat16)
        wu = wubuf[slot].astype(jnp.bfloat16)
        wd = wdbuf[slot].astype(jnp.bfloat16)
        g = lax.dot_general(xb, wg, (((1,), (1,)), ((), ())),
                            preferred_element_type=jnp.float32)
        u = lax.dot_general(xb, wu, (((1,), (1,)), ((), ())),
                            preferred_element_type=jnp.float32)
        h = (g * jax.nn.sigmoid(g) * u).astype(jnp.bfloat16)
        y = lax.dot_general(h, wd, (((1,), (1,)), ((), ())),
                            preferred_element_type=jnp.float32)
        y_ref[...] = y * ws_ref[:, 0:1]


def _moe(block_expert, xs, ws, w_gate, w_up, w_down):
    grid_spec = pltpu.PrefetchScalarGridSpec(
        num_scalar_prefetch=1,
        grid=(NB,),
        in_specs=[
            pl.BlockSpec((BLK, D), lambda i, be: (i, 0)),
            pl.BlockSpec((BLK, 128), lambda i, be: (i, 0)),
            pl.BlockSpec(memory_space=pl.ANY),
            pl.BlockSpec(memory_space=pl.ANY),
            pl.BlockSpec(memory_space=pl.ANY),
        ],
        out_specs=pl.BlockSpec((BLK, D), lambda i, be: (i, 0)),
        scratch_shapes=[
            pltpu.VMEM((2, F, D), jnp.float32),
            pltpu.VMEM((2, F, D), jnp.float32),
            pltpu.VMEM((2, D, F), jnp.float32),
            pltpu.SemaphoreType.DMA((2,)),
        ],
    )
    return pl.pallas_call(
        _moe_body,
        grid_spec=grid_spec,
        out_shape=jax.ShapeDtypeStruct((NROWS, D), jnp.float32),
    )(block_expert, xs, ws, w_gate, w_up, w_down)


def _shared_body(x_ref, sg_ref, su_ref, sd_ref, o_ref):
    xb = x_ref[...]
    sg = sg_ref[...].astype(jnp.bfloat16)
    su = su_ref[...].astype(jnp.bfloat16)
    sd = sd_ref[...].astype(jnp.bfloat16)
    g = lax.dot_general(xb, sg, (((1,), (1,)), ((), ())),
                        preferred_element_type=jnp.float32)
    u = lax.dot_general(xb, su, (((1,), (1,)), ((), ())),
                        preferred_element_type=jnp.float32)
    h = (g * jax.nn.sigmoid(g) * u).astype(jnp.bfloat16)
    part = lax.dot_general(h, sd, (((1,), (1,)), ((), ())),
                           preferred_element_type=jnp.float32)

    @pl.when(pl.program_id(0) == 0)
    def _():
        o_ref[...] = part

    @pl.when(pl.program_id(0) != 0)
    def _():
        o_ref[...] += part


def _shared(x_bf, shared_gate, shared_up, shared_down):
    nfb = FS // FSB
    return pl.pallas_call(
        _shared_body,
        grid=(nfb,),
        in_specs=[
            pl.BlockSpec((T, D), lambda i: (0, 0)),
            pl.BlockSpec((FSB, D), lambda i: (i, 0)),
            pl.BlockSpec((FSB, D), lambda i: (i, 0)),
            pl.BlockSpec((D, FSB), lambda i: (0, i)),
        ],
        out_specs=pl.BlockSpec((T, D), lambda i: (0, 0)),
        out_shape=jax.ShapeDtypeStruct((T, D), jnp.float32),
    )(x_bf, shared_gate, shared_up, shared_down)


_NW = 32
_DCH = 32


def _dispatch(x, pos0_2d, pos1_2d, w0_rows, w1_rows):
    mesh = plsc.VectorSubcoreMesh(core_axis_name="c", subcore_axis_name="s")
    per_w = T // _NW
    nch = per_w // _DCH

    @functools.partial(
        pl.kernel,
        out_type=(jax.ShapeDtypeStruct((NROWS, D), jnp.float32),
                  jax.ShapeDtypeStruct((NROWS, 128), jnp.float32)),
        mesh=mesh,
        scratch_types=[
            pltpu.VMEM((nch, _DCH), jnp.int32),
            pltpu.VMEM((nch, _DCH), jnp.int32),
            pltpu.VMEM((_DCH, D), jnp.float32),
            pltpu.VMEM((_DCH, 128), jnp.float32),
            pltpu.VMEM((_DCH, 128), jnp.float32),
        ],
    )
    def disp(x_hbm, p0_hbm, p1_hbm, w0_hbm, w1_hbm, xs_hbm, ws_hbm,
             p0buf, p1buf, xbuf, w0buf, w1buf):
        wid = lax.axis_index("s") * 2 + lax.axis_index("c")
        pltpu.sync_copy(p0_hbm.at[pl.ds(wid * nch, nch)], p0buf)
        pltpu.sync_copy(p1_hbm.at[pl.ds(wid * nch, nch)], p1buf)
        base = wid * per_w
        for c in range(nch):
            off = base + c * _DCH
            pltpu.sync_copy(x_hbm.at[pl.ds(off, _DCH)], xbuf)
            pltpu.sync_copy(w0_hbm.at[pl.ds(off, _DCH)], w0buf)
            pltpu.sync_copy(w1_hbm.at[pl.ds(off, _DCH)], w1buf)
            pltpu.sync_copy(xbuf, xs_hbm.at[p0buf.at[c]])
            pltpu.sync_copy(xbuf, xs_hbm.at[p1buf.at[c]])
            pltpu.sync_copy(w0buf, ws_hbm.at[p0buf.at[c]])
            pltpu.sync_copy(w1buf, ws_hbm.at[p1buf.at[c]])

    return disp(x, pos0_2d, pos1_2d, w0_rows, w1_rows)


_CCH = 8


def _combine(ys, sh, pos0, pos1):
    mesh = plsc.VectorSubcoreMesh(core_axis_name="c", subcore_axis_name="s")
    per_w = T // _NW
    nch = per_w // _CCH

    @functools.partial(
        pl.kernel,
        out_type=jax.ShapeDtypeStruct((T, D), jnp.float32),
        mesh=mesh,
        scratch_types=[
            pltpu.VMEM((per_w,), jnp.int32),
            pltpu.VMEM((per_w,), jnp.int32),
            [pltpu.VMEM((_CCH, D), jnp.float32) for _ in range(2)],
            [pltpu.VMEM((_CCH, D), jnp.float32) for _ in range(2)],
            [pltpu.VMEM((_CCH, D), jnp.float32) for _ in range(2)],
            [pltpu.SemaphoreType.DMA for _ in range(2)],
            pltpu.SemaphoreType.DMA,
        ],
    )
    def comb(ys_hbm, sh_hbm, p0_hbm, p1_hbm, o_hbm, i0buf, i1buf,
             y0bufs, y1bufs, obufs, gsems, osem):
        wid = lax.axis_index("s") * 2 + lax.axis_index("c")
        base = wid * per_w
        pltpu.sync_copy(p0_hbm.at[pl.ds(base, per_w)], i0buf)
        pltpu.sync_copy(p1_hbm.at[pl.ds(base, per_w)], i1buf)

        pend = {}
        owrite = {}

        def start(c, b):
            off = base + c * _CCH
            ii = pl.ds(c * _CCH, _CCH)
            pend[b] = (
                pltpu.async_copy(ys_hbm.at[i0buf.at[ii]], y0bufs[b],
                                 gsems[b]),
                pltpu.async_copy(ys_hbm.at[i1buf.at[ii]], y1bufs[b],
                                 gsems[b]),
                pltpu.async_copy(sh_hbm.at[pl.ds(off, _CCH)], obufs[b],
                                 gsems[b]),
            )

        start(0, 0)
        for c in range(nch):
            b = c % 2
            if c + 1 < nch:
                if (1 - b) in owrite:
                    owrite.pop(1 - b).wait()
                start(c + 1, 1 - b)
            for d in pend.pop(b):
                d.wait()
            y0buf, y1buf, obuf = y0bufs[b], y1bufs[b], obufs[b]

            @pl.loop(0, _CCH)
            def _(r):
                @pl.loop(0, D, step=16)
                def _(j):
                    slc = (r, pl.ds(j, 16))
                    obuf[slc] = obuf[slc] + y0buf[slc] + y1buf[slc]

            owrite[b] = pltpu.async_copy(
                obuf, o_hbm.at[pl.ds(base + c * _CCH, _CCH)], osem)
        for d in owrite.values():
            d.wait()

    return comb(ys, sh, pos0, pos1)


def kernel(x, router_w, w_gate, w_up, w_down, shared_gate, shared_up,
           shared_down):
    w, pos, block_expert, w0_rows, w1_rows = _router(x, router_w)

    pos0 = pos[:, 0]
    pos1 = pos[:, 1]
    xs, ws = _dispatch(x, pos0.reshape(T // _DCH, _DCH),
                       pos1.reshape(T // _DCH, _DCH), w0_rows, w1_rows)

    sh = _shared(x.astype(jnp.bfloat16), shared_gate, shared_up, shared_down)
    ys = _moe(block_expert, xs, ws, w_gate, w_up, w_down)

    return _combine(ys, sh, pos0, pos1)
